# baseline (speedup 1.0000x reference)
import functools

import jax
import jax.numpy as jnp
from jax import lax
from jax.experimental import pallas as pl
from jax.experimental.pallas import tpu as pltpu
from jax.experimental.pallas import tpu_sc as plsc

D_MODEL = 768
B = 4
S = 2048
SEG_BOUNDARY = S // 2 + 1

NC = 2
NS = 16
NW = NC * NS
LANES = 16
NCHUNK_VECS = D_MODEL // LANES

TOTAL_ROWS = B * S
ROWS_PER_W = TOTAL_ROWS // NW
CHUNK = 64
NCHUNKS = ROWS_PER_W // CHUNK


_GATHER_DNUMS = lax.GatherDimensionNumbers(
    offset_dims=(), collapsed_slice_dims=(0,), start_index_map=(0,))


def _lane_shuffle(x, perm):
    return lax.gather(x, perm[:, None], _GATHER_DNUMS, slice_sizes=(1,),
                      mode=lax.GatherScatterMode.PROMISE_IN_BOUNDS)


def _lane_sum(x):
    lanes = lax.iota(jnp.int32, LANES)
    for k in (8, 4, 2, 1):
        x = x + _lane_shuffle(x, lanes ^ k)
    return x


def _rsqrt(v):
    i = lax.bitcast_convert_type(v, jnp.int32)
    i = jnp.int32(0x5F3759DF) - (i >> 1)
    y = lax.bitcast_convert_type(i, jnp.float32)
    for _ in range(3):
        y = y * (jnp.float32(1.5) - jnp.float32(0.5) * v * y * y)
    return y


def _body(idx_hbm, tab_hbm, seg_hbm, gam_hbm, bet_hbm, pos_hbm, out_hbm,
          idx_v, x_v, pos_v, seg_v, gam_v, bet_v, sem):
    w = lax.axis_index("s") * NC + lax.axis_index("c")

    pltpu.sync_copy(seg_hbm, seg_v)
    pltpu.sync_copy(gam_hbm, gam_v)
    pltpu.sync_copy(bet_hbm, bet_v)

    def chunk_body(g, _):
        flat0 = w * ROWS_PER_W + g * CHUNK
        p0 = lax.rem(w, 8) * ROWS_PER_W + g * CHUNK

        pltpu.sync_copy(idx_hbm.at[pl.ds(flat0, CHUNK)], idx_v)
        pltpu.async_copy(tab_hbm.at[idx_v], x_v, sem).wait()
        pltpu.sync_copy(pos_hbm.at[pl.ds(p0, CHUNK)], pos_v)

        def row_body(r, _):
            p = p0 + r
            first_seg = p < SEG_BOUNDARY
            acc = jnp.zeros((LANES,), jnp.float32)
            acc2 = jnp.zeros((LANES,), jnp.float32)
            for j in range(NCHUNK_VECS):
                sl = pl.ds(j * LANES, LANES)
                s0 = seg_v[0, sl]
                s1 = seg_v[1, sl]
                x = x_v[r, sl] + pos_v[r, sl] + jnp.where(first_seg, s0, s1)
                x_v[r, sl] = x
                acc = acc + x
                acc2 = acc2 + x * x
            mean = _lane_sum(acc) * jnp.float32(1.0 / D_MODEL)
            var = _lane_sum(acc2) * jnp.float32(1.0 / D_MODEL) - mean * mean
            inv = _rsqrt(var + jnp.float32(1e-5))
            for j in range(NCHUNK_VECS):
                sl = pl.ds(j * LANES, LANES)
                x_v[r, sl] = (x_v[r, sl] - mean) * inv * gam_v[sl] + bet_v[sl]
            return 0

        lax.fori_loop(0, CHUNK, row_body, 0)
        pltpu.sync_copy(x_v, out_hbm.at[pl.ds(flat0, CHUNK)])
        return 0

    lax.fori_loop(0, NCHUNKS, chunk_body, 0)


@jax.jit
def _run(idx_flat, token_table, segment_table, ln_gamma, ln_beta, pos_enc):
    mesh = plsc.VectorSubcoreMesh(core_axis_name="c", subcore_axis_name="s")
    f = functools.partial(
        pl.kernel,
        out_type=jax.ShapeDtypeStruct((TOTAL_ROWS, D_MODEL), jnp.float32),
        mesh=mesh,
        scratch_types=[
            pltpu.VMEM((CHUNK,), jnp.int32),
            pltpu.VMEM((CHUNK, D_MODEL), jnp.float32),
            pltpu.VMEM((CHUNK, D_MODEL), jnp.float32),
            pltpu.VMEM((2, D_MODEL), jnp.float32),
            pltpu.VMEM((D_MODEL,), jnp.float32),
            pltpu.VMEM((D_MODEL,), jnp.float32),
            pltpu.SemaphoreType.DMA,
        ],
    )(_body)
    return f(idx_flat, token_table, segment_table, ln_gamma, ln_beta, pos_enc)


def kernel(idx, token_table, segment_table, ln_gamma, ln_beta, pos_enc):
    idx_flat = idx.reshape(-1).astype(jnp.int32)
    out = _run(idx_flat, token_table, segment_table, ln_gamma, ln_beta,
               pos_enc[:S])
    return out.reshape(idx.shape[0], idx.shape[1], D_MODEL)
